# baseline (speedup 1.0000x reference)
import functools

import jax
import jax.numpy as jnp
from jax import lax
from jax.experimental import pallas as pl
from jax.experimental.pallas import tpu as pltpu
from jax.experimental.pallas import tpu_sc as plsc

NN = 10000
DD = 128
EE = 320000
NC = 2
NS = 16
NW = NC * NS
CHUNK = 40
CH = 256
NCHUNK = NW * CH
EPAD = NCHUNK * CHUNK
NPAD = 10112
RPT = NPAD // NS
ZR = 8
NBUF = 4
RB = NPAD // 8


def _sc_mesh():
    return plsc.VectorSubcoreMesh(
        core_axis_name="c", subcore_axis_name="s", num_cores=NC, num_subcores=NS
    )


def _sc_count(idx2d):
    @functools.partial(
        pl.kernel,
        out_type=jax.ShapeDtypeStruct((NC, NPAD, 16), jnp.float32),
        mesh=_sc_mesh(),
        compiler_params=pltpu.CompilerParams(use_tc_tiling_on_sc=False),
        scratch_types=[
            pltpu.VMEM((CH, CHUNK), jnp.int32),
            pltpu.VMEM((CHUNK, 16), jnp.float32),
            pltpu.VMEM((ZR, 16), jnp.float32),
            pltpu.VMEM_SHARED((NPAD, 16), jnp.float32),
        ],
    )
    def cnt_kernel(idx_hbm, cnt_hbm, idx_v, ones_v, zero_v, acc):
        c = lax.axis_index("c")
        s = lax.axis_index("s")
        wid = s * NC + c
        ones16 = jnp.ones((16,), jnp.float32)
        zeros16 = jnp.zeros((16,), jnp.float32)

        def fill_ones(i, carry):
            ones_v[i] = ones16
            return carry

        lax.fori_loop(0, CHUNK, fill_ones, 0)

        def fill_zeros(i, carry):
            zero_v[i] = zeros16
            return carry

        lax.fori_loop(0, ZR, fill_zeros, 0)

        base = s * RPT

        def zinit(i, carry):
            pltpu.sync_copy(zero_v, acc.at[pl.ds(base + i * ZR, ZR)])
            return carry

        lax.fori_loop(0, RPT // ZR, zinit, 0)
        plsc.subcore_barrier()

        pltpu.sync_copy(idx_hbm.at[pl.ds(wid * CH, CH)], idx_v)

        def body(j, carry):
            pltpu.sync_copy(ones_v, acc.at[idx_v.at[j]], add=True)
            return carry

        lax.fori_loop(0, CH, body, 0)
        plsc.subcore_barrier()

        pltpu.sync_copy(acc.at[pl.ds(base, RPT)],
                        cnt_hbm.at[c, pl.ds(base, RPT)])

    return cnt_kernel(idx2d)


def _sc_agg(y, src3d, dst3d):
    @functools.partial(
        pl.kernel,
        out_type=jax.ShapeDtypeStruct((NC, NPAD, DD), jnp.float32),
        mesh=_sc_mesh(),
        compiler_params=pltpu.CompilerParams(use_tc_tiling_on_sc=False),
        scratch_types=[
            pltpu.VMEM((CH, CHUNK), jnp.int32),
            pltpu.VMEM((CH, CHUNK), jnp.int32),
            pltpu.VMEM_SHARED((NPAD, DD), jnp.float32),
            [pltpu.SemaphoreType.DMA] * NBUF,
            [pltpu.SemaphoreType.DMA] * NBUF,
        ],
    )
    def agg_kernel(y_hbm, src_hbm, dst_hbm, p_hbm, src_v, dst_v, acc,
                   gsems, ssems):
        c = lax.axis_index("c")
        s = lax.axis_index("s")
        wid = s * NC + c
        zeros16 = jnp.zeros((16,), jnp.float32)

        def scoped(*rows):
            def fz(i, carry):
                rows[0][i // 8, pl.ds((i % 8) * 16, 16)] = zeros16
                return carry

            lax.fori_loop(0, ZR * 8, fz, 0)

            base = s * RPT
            zsrc = rows[0].at[pl.ds(0, ZR)]

            def zinit(i, carry):
                pltpu.sync_copy(zsrc, acc.at[pl.ds(base + i * ZR, ZR)])
                return carry

            lax.fori_loop(0, RPT // ZR, zinit, 0)
            plsc.subcore_barrier()

            pltpu.sync_copy(src_hbm.at[wid], src_v)
            pltpu.sync_copy(dst_hbm.at[wid], dst_v)

            pltpu.async_copy(y_hbm.at[src_v.at[0]], rows[0], gsems[0])
            pltpu.async_copy(y_hbm.at[src_v.at[1]], rows[1], gsems[1])

            def body(g, carry):
                for b in range(NBUF):
                    j = g * NBUF + b
                    pltpu.make_async_copy(y_hbm.at[src_v.at[j]], rows[b],
                                          gsems[b]).wait()
                    pltpu.async_copy(rows[b], acc.at[dst_v.at[j]], ssems[b],
                                     add=True)
                    bn = (b + 2) % NBUF

                    @pl.when(j + 2 < CH)
                    def _():
                        @pl.when(j >= 2)
                        def _():
                            pltpu.make_async_copy(
                                rows[bn], acc.at[dst_v.at[j - 2]],
                                ssems[bn]).wait()

                        pltpu.async_copy(y_hbm.at[src_v.at[j + 2]], rows[bn],
                                         gsems[bn])

                return carry

            lax.fori_loop(0, CH // NBUF, body, 0)
            for j in range(CH - 4, CH):
                b = j % NBUF
                pltpu.make_async_copy(rows[b], acc.at[dst_v.at[j]],
                                      ssems[b]).wait()
            plsc.subcore_barrier()

            pltpu.sync_copy(acc.at[pl.ds(base, RPT)],
                            p_hbm.at[c, pl.ds(base, RPT)])

        pl.run_scoped(
            scoped, *([pltpu.VMEM((CHUNK, DD), jnp.float32)] * NBUF)
        )

    return agg_kernel(y, src3d, dst3d)


def _tc_norm_matmul(xp, W1, cs0, cs1, cd0, cd1):
    def body(x_ref, w_ref, cs0_ref, cs1_ref, cd0_ref, cd1_ref,
             y_ref, so_ref, si_ref):
        deg_o = jnp.maximum(cs0_ref[:, 0:1] + cs1_ref[:, 0:1], 1.0)
        deg_i = jnp.maximum(cd0_ref[:, 0:1] + cd1_ref[:, 0:1], 1.0)
        so = jnp.broadcast_to(lax.rsqrt(deg_o), (RB, DD))
        si = jnp.broadcast_to(lax.rsqrt(deg_i), (RB, DD))
        y = jnp.dot(x_ref[...], w_ref[...], preferred_element_type=jnp.float32,
                    precision=lax.Precision.HIGHEST)
        y_ref[...] = y * so
        so_ref[...] = so
        si_ref[...] = si

    row = pl.BlockSpec((RB, DD), lambda i: (i, 0))
    cnt = pl.BlockSpec((RB, 16), lambda i: (i, 0))
    full = pl.BlockSpec((DD, DD), lambda i: (0, 0))
    shape = jax.ShapeDtypeStruct((NPAD, DD), jnp.float32)
    return pl.pallas_call(
        body,
        grid=(NPAD // RB,),
        in_specs=[row, full, cnt, cnt, cnt, cnt],
        out_specs=[row, row, row],
        out_shape=[shape, shape, shape],
    )(xp, W1, cs0, cs1, cd0, cd1)


def _tc_mid(p0, p1, si, so, b1, W2):
    def body(p0_ref, p1_ref, si_ref, so_ref, b_ref, w_ref, y_ref):
        agg = (p0_ref[...] + p1_ref[...]) * si_ref[...]
        h = agg + b_ref[...]
        h = jnp.where(h >= 0, h, h * jnp.float32(0.01))
        y = jnp.dot(h, w_ref[...], preferred_element_type=jnp.float32,
                    precision=lax.Precision.HIGHEST)
        y_ref[...] = y * so_ref[...]

    row = pl.BlockSpec((RB, DD), lambda i: (i, 0))
    bias = pl.BlockSpec((1, DD), lambda i: (0, 0))
    full = pl.BlockSpec((DD, DD), lambda i: (0, 0))
    return pl.pallas_call(
        body,
        grid=(NPAD // RB,),
        in_specs=[row, row, row, row, bias, full],
        out_specs=row,
        out_shape=jax.ShapeDtypeStruct((NPAD, DD), jnp.float32),
    )(p0, p1, si, so, b1, W2)


def _tc_final(p0, p1, si, b2):
    def body(p0_ref, p1_ref, si_ref, b_ref, y_ref):
        y_ref[...] = (p0_ref[...] + p1_ref[...]) * si_ref[...] + b_ref[...]

    row = pl.BlockSpec((RB, DD), lambda i: (i, 0))
    bias = pl.BlockSpec((1, DD), lambda i: (0, 0))
    return pl.pallas_call(
        body,
        grid=(NPAD // RB,),
        in_specs=[row, row, row, bias],
        out_specs=row,
        out_shape=jax.ShapeDtypeStruct((NPAD, DD), jnp.float32),
    )(p0, p1, si, b2)


def kernel(x, edge_index, W1, b1, W2, b2):
    src = edge_index[0]
    dst = edge_index[1]
    fill = jnp.full((EPAD - EE,), NN, dtype=jnp.int32)
    srcp = jnp.concatenate([src, fill])
    dstp = jnp.concatenate([dst, fill])
    src2d = srcp.reshape(NCHUNK, CHUNK)
    dst2d = dstp.reshape(NCHUNK, CHUNK)
    src3d = srcp.reshape(NW, CH, CHUNK)
    dst3d = dstp.reshape(NW, CH, CHUNK)
    xp = jnp.zeros((NPAD, DD), jnp.float32).at[:NN].set(x)

    cnt_s = _sc_count(src2d)
    cnt_d = _sc_count(dst2d)
    y1, so, si = _tc_norm_matmul(xp, W1, cnt_s[0], cnt_s[1],
                                 cnt_d[0], cnt_d[1])
    p1 = _sc_agg(y1, src3d, dst3d)
    y2 = _tc_mid(p1[0], p1[1], si, so, b1.reshape(1, DD), W2)
    p2 = _sc_agg(y2, src3d, dst3d)
    out = _tc_final(p2[0], p2[1], si, b2.reshape(1, DD))
    return out[:NN]


# speedup vs baseline: 1.0122x; 1.0122x over previous
import functools

import jax
import jax.numpy as jnp
from jax import lax
from jax.experimental import pallas as pl
from jax.experimental.pallas import tpu as pltpu
from jax.experimental.pallas import tpu_sc as plsc

NN = 10000
DD = 128
EE = 320000
NC = 2
NS = 16
NW = NC * NS
CHUNK = 80
CH = 128
NCHUNK = NW * CH
EPAD = NCHUNK * CHUNK
NPAD = 10112
RPT = NPAD // NS
ZR = 8
NBUF = 2
RB = NPAD // 8


def _sc_mesh():
    return plsc.VectorSubcoreMesh(
        core_axis_name="c", subcore_axis_name="s", num_cores=NC, num_subcores=NS
    )


def _sc_count(idx2d):
    @functools.partial(
        pl.kernel,
        out_type=jax.ShapeDtypeStruct((NC, NPAD, 16), jnp.float32),
        mesh=_sc_mesh(),
        compiler_params=pltpu.CompilerParams(use_tc_tiling_on_sc=False),
        scratch_types=[
            pltpu.VMEM((CH, CHUNK), jnp.int32),
            pltpu.VMEM((CHUNK, 16), jnp.float32),
            pltpu.VMEM((ZR, 16), jnp.float32),
            pltpu.VMEM_SHARED((NPAD, 16), jnp.float32),
        ],
    )
    def cnt_kernel(idx_hbm, cnt_hbm, idx_v, ones_v, zero_v, acc):
        c = lax.axis_index("c")
        s = lax.axis_index("s")
        wid = s * NC + c
        ones16 = jnp.ones((16,), jnp.float32)
        zeros16 = jnp.zeros((16,), jnp.float32)

        def fill_ones(i, carry):
            ones_v[i] = ones16
            return carry

        lax.fori_loop(0, CHUNK, fill_ones, 0)

        def fill_zeros(i, carry):
            zero_v[i] = zeros16
            return carry

        lax.fori_loop(0, ZR, fill_zeros, 0)

        base = s * RPT

        def zinit(i, carry):
            pltpu.sync_copy(zero_v, acc.at[pl.ds(base + i * ZR, ZR)])
            return carry

        lax.fori_loop(0, RPT // ZR, zinit, 0)
        plsc.subcore_barrier()

        pltpu.sync_copy(idx_hbm.at[pl.ds(wid * CH, CH)], idx_v)

        def body(j, carry):
            pltpu.sync_copy(ones_v, acc.at[idx_v.at[j]], add=True)
            return carry

        lax.fori_loop(0, CH, body, 0)
        plsc.subcore_barrier()

        pltpu.sync_copy(acc.at[pl.ds(base, RPT)],
                        cnt_hbm.at[c, pl.ds(base, RPT)])

    return cnt_kernel(idx2d)


def _sc_agg(y, src3d, dst3d):
    @functools.partial(
        pl.kernel,
        out_type=jax.ShapeDtypeStruct((NC, NPAD, DD), jnp.float32),
        mesh=_sc_mesh(),
        compiler_params=pltpu.CompilerParams(use_tc_tiling_on_sc=False),
        scratch_types=[
            pltpu.VMEM((CH, CHUNK), jnp.int32),
            pltpu.VMEM((CH, CHUNK), jnp.int32),
            pltpu.VMEM_SHARED((NPAD, DD), jnp.float32),
            [pltpu.SemaphoreType.DMA] * NBUF,
            [pltpu.SemaphoreType.DMA] * NBUF,
        ],
    )
    def agg_kernel(y_hbm, src_hbm, dst_hbm, p_hbm, src_v, dst_v, acc,
                   gsems, ssems):
        c = lax.axis_index("c")
        s = lax.axis_index("s")
        wid = s * NC + c
        zeros16 = jnp.zeros((16,), jnp.float32)

        def scoped(*rows):
            def fz(i, carry):
                rows[0][i // 8, pl.ds((i % 8) * 16, 16)] = zeros16
                return carry

            lax.fori_loop(0, ZR * 8, fz, 0)

            base = s * RPT
            zsrc = rows[0].at[pl.ds(0, ZR)]

            def zinit(i, carry):
                pltpu.sync_copy(zsrc, acc.at[pl.ds(base + i * ZR, ZR)])
                return carry

            lax.fori_loop(0, RPT // ZR, zinit, 0)
            plsc.subcore_barrier()

            pltpu.sync_copy(src_hbm.at[wid], src_v)
            pltpu.sync_copy(dst_hbm.at[wid], dst_v)

            pltpu.async_copy(y_hbm.at[src_v.at[0]], rows[0], gsems[0])
            pltpu.async_copy(y_hbm.at[src_v.at[1]], rows[1], gsems[1])

            def body(g, carry):
                for b in range(NBUF):
                    j = g * NBUF + b
                    pltpu.make_async_copy(y_hbm.at[src_v.at[j]], rows[b],
                                          gsems[b]).wait()
                    if NBUF == 2:
                        pltpu.sync_copy(rows[b], acc.at[dst_v.at[j]],
                                        add=True)

                        @pl.when(j + 2 < CH)
                        def _():
                            pltpu.async_copy(y_hbm.at[src_v.at[j + 2]],
                                             rows[b], gsems[b])
                    else:
                        pltpu.async_copy(rows[b], acc.at[dst_v.at[j]],
                                         ssems[b], add=True)
                        bn = (b + 2) % NBUF

                        @pl.when(j + 2 < CH)
                        def _():
                            @pl.when(j >= 2)
                            def _():
                                pltpu.make_async_copy(
                                    rows[bn], acc.at[dst_v.at[j - 2]],
                                    ssems[bn]).wait()

                            pltpu.async_copy(y_hbm.at[src_v.at[j + 2]],
                                             rows[bn], gsems[bn])

                return carry

            lax.fori_loop(0, CH // NBUF, body, 0)
            if NBUF != 2:
                for j in range(CH - 4, CH):
                    b = j % NBUF
                    pltpu.make_async_copy(rows[b], acc.at[dst_v.at[j]],
                                          ssems[b]).wait()
            plsc.subcore_barrier()

            pltpu.sync_copy(acc.at[pl.ds(base, RPT)],
                            p_hbm.at[c, pl.ds(base, RPT)])

        pl.run_scoped(
            scoped, *([pltpu.VMEM((CHUNK, DD), jnp.float32)] * NBUF)
        )

    return agg_kernel(y, src3d, dst3d)


def _tc_norm_matmul(xp, W1, cs0, cs1, cd0, cd1):
    def body(x_ref, w_ref, cs0_ref, cs1_ref, cd0_ref, cd1_ref,
             y_ref, so_ref, si_ref):
        deg_o = jnp.maximum(cs0_ref[:, 0:1] + cs1_ref[:, 0:1], 1.0)
        deg_i = jnp.maximum(cd0_ref[:, 0:1] + cd1_ref[:, 0:1], 1.0)
        so = jnp.broadcast_to(lax.rsqrt(deg_o), (RB, DD))
        si = jnp.broadcast_to(lax.rsqrt(deg_i), (RB, DD))
        y = jnp.dot(x_ref[...], w_ref[...], preferred_element_type=jnp.float32,
                    precision=lax.Precision.HIGHEST)
        y_ref[...] = y * so
        so_ref[...] = so
        si_ref[...] = si

    row = pl.BlockSpec((RB, DD), lambda i: (i, 0))
    cnt = pl.BlockSpec((RB, 16), lambda i: (i, 0))
    full = pl.BlockSpec((DD, DD), lambda i: (0, 0))
    shape = jax.ShapeDtypeStruct((NPAD, DD), jnp.float32)
    return pl.pallas_call(
        body,
        grid=(NPAD // RB,),
        in_specs=[row, full, cnt, cnt, cnt, cnt],
        out_specs=[row, row, row],
        out_shape=[shape, shape, shape],
    )(xp, W1, cs0, cs1, cd0, cd1)


def _tc_mid(p0, p1, si, so, b1, W2):
    def body(p0_ref, p1_ref, si_ref, so_ref, b_ref, w_ref, y_ref):
        agg = (p0_ref[...] + p1_ref[...]) * si_ref[...]
        h = agg + b_ref[...]
        h = jnp.where(h >= 0, h, h * jnp.float32(0.01))
        y = jnp.dot(h, w_ref[...], preferred_element_type=jnp.float32,
                    precision=lax.Precision.HIGHEST)
        y_ref[...] = y * so_ref[...]

    row = pl.BlockSpec((RB, DD), lambda i: (i, 0))
    bias = pl.BlockSpec((1, DD), lambda i: (0, 0))
    full = pl.BlockSpec((DD, DD), lambda i: (0, 0))
    return pl.pallas_call(
        body,
        grid=(NPAD // RB,),
        in_specs=[row, row, row, row, bias, full],
        out_specs=row,
        out_shape=jax.ShapeDtypeStruct((NPAD, DD), jnp.float32),
    )(p0, p1, si, so, b1, W2)


def _tc_final(p0, p1, si, b2):
    def body(p0_ref, p1_ref, si_ref, b_ref, y_ref):
        y_ref[...] = (p0_ref[...] + p1_ref[...]) * si_ref[...] + b_ref[...]

    row = pl.BlockSpec((RB, DD), lambda i: (i, 0))
    bias = pl.BlockSpec((1, DD), lambda i: (0, 0))
    return pl.pallas_call(
        body,
        grid=(NPAD // RB,),
        in_specs=[row, row, row, bias],
        out_specs=row,
        out_shape=jax.ShapeDtypeStruct((NPAD, DD), jnp.float32),
    )(p0, p1, si, b2)


def kernel(x, edge_index, W1, b1, W2, b2):
    src = edge_index[0]
    dst = edge_index[1]
    fill = jnp.full((EPAD - EE,), NN, dtype=jnp.int32)
    srcp = jnp.concatenate([src, fill])
    dstp = jnp.concatenate([dst, fill])
    src2d = srcp.reshape(NCHUNK, CHUNK)
    dst2d = dstp.reshape(NCHUNK, CHUNK)
    src3d = srcp.reshape(NW, CH, CHUNK)
    dst3d = dstp.reshape(NW, CH, CHUNK)
    xp = jnp.zeros((NPAD, DD), jnp.float32).at[:NN].set(x)

    cnt_s = _sc_count(src2d)
    cnt_d = _sc_count(dst2d)
    y1, so, si = _tc_norm_matmul(xp, W1, cnt_s[0], cnt_s[1],
                                 cnt_d[0], cnt_d[1])
    p1 = _sc_agg(y1, src3d, dst3d)
    y2 = _tc_mid(p1[0], p1[1], si, so, b1.reshape(1, DD), W2)
    p2 = _sc_agg(y2, src3d, dst3d)
    out = _tc_final(p2[0], p2[1], si, b2.reshape(1, DD))
    return out[:NN]
